# baseline (speedup 1.0000x reference)
import jax
import jax.numpy as jnp
from jax import lax
from jax.experimental import pallas as pl
from jax.experimental.pallas import tpu as pltpu
from jax.experimental.pallas import tpu_sc as plsc

NC = 2
NS = 16
NW = NC * NS

D = 32
NR = 16384
NJ = 50
IW = 128
KU = 4
CHUNK = KU * IW


def _gather_body(table_hbm, idxt_hbm, out_hbm, idx_v, rows_v, buf2, gsem, wsem):
    c = lax.axis_index("c")
    s = lax.axis_index("s")
    wid = s * NC + c
    i0 = wid * CHUNK

    def fire(j, b):
        for k in range(KU):
            pltpu.sync_copy(
                idxt_hbm.at[j, pl.ds(i0 + k * IW, IW)], idx_v.at[b, k]
            )
            pltpu.async_copy(
                table_hbm.at[idx_v.at[b, k]],
                rows_v.at[b, k],
                gsem.at[b],
            )

    def wait_gathers(b):
        for k in range(KU):
            pltpu.make_async_copy(
                table_hbm.at[pl.ds(0, IW)], rows_v.at[b, k], gsem.at[b]
            ).wait()

    def wait_writeback(b):
        for k in range(KU):
            pltpu.make_async_copy(
                buf2.at[b, k], out_hbm.at[0, :, k], wsem.at[b]
            ).wait()

    fire(0, 0)

    def body(j, carry):
        b = lax.rem(j, 2)
        nb = 1 - b

        @pl.when(j + 1 < NJ)
        def _prefetch():
            fire(j + 1, nb)

        wait_gathers(b)

        @pl.when(j >= 2)
        def _drain():
            wait_writeback(b)

        def tk(k, carry2):
            iota = lax.iota(jnp.int32, 16)
            rv = rows_v.at[b, k]
            for dt in range(D // 8):
                for dr in range(8):
                    col = jnp.full((16,), dt * 8 + dr, jnp.int32)
                    for q in range(IW // 16):
                        vals = plsc.load_gather(rv, [q * 16 + iota, col])
                        buf2[b, k, dt, pl.ds(dr * IW + q * 16, 16)] = vals
            return carry2

        lax.fori_loop(0, KU, tk, 0)

        for k in range(KU):
            pltpu.async_copy(
                buf2.at[b, k],
                out_hbm.at[j, :, wid * KU + k],
                wsem.at[b],
            )
        return carry

    lax.fori_loop(0, NJ, body, 0)
    wait_writeback(0)
    wait_writeback(1)


@jax.jit
def _gather(data, idxt):
    mesh = plsc.VectorSubcoreMesh(
        core_axis_name="c", subcore_axis_name="s",
        num_cores=NC, num_subcores=NS,
    )
    k = pl.kernel(
        _gather_body,
        out_type=jax.ShapeDtypeStruct((NJ, D // 8, NR // IW, 8 * IW),
                                      jnp.float32),
        mesh=mesh,
        scratch_types=[
            pltpu.VMEM((2, KU, IW), jnp.int32),
            pltpu.VMEM((2, KU, IW, D), jnp.float32),
            pltpu.VMEM((2, KU, D // 8, 8 * IW), jnp.float32),
            pltpu.SemaphoreType.DMA((2,)),
            pltpu.SemaphoreType.DMA((2,)),
        ],
        compiler_params=pltpu.CompilerParams(
            use_tc_tiling_on_sc=False, needs_layout_passes=False
        ),
    )
    out4 = k(data, idxt)
    out6 = out4.reshape(NJ, D // 8, NR // IW, 8, IW)
    return jnp.transpose(out6, (2, 4, 0, 1, 3)).reshape(NR, NJ, D)


def kernel(data, indices):
    return _gather(data, indices.astype(jnp.int32).T)


# speedup vs baseline: 1.1346x; 1.1346x over previous
import jax
import jax.numpy as jnp
from jax import lax
from jax.experimental import pallas as pl
from jax.experimental.pallas import tpu as pltpu
from jax.experimental.pallas import tpu_sc as plsc

NC = 2
NS = 16
NW = NC * NS

D = 32
NR = 16384
NJ = 50
IW = 128
KU = 4
CHUNK = KU * IW


def _gather_body(table_hbm, idxt_hbm, out_hbm, idx_v, rows_v, buf2, gsem, wsem):
    c = lax.axis_index("c")
    s = lax.axis_index("s")
    wid = s * NC + c
    i0 = wid * CHUNK

    def fire(j, b):
        for k in range(KU):
            pltpu.sync_copy(
                idxt_hbm.at[j, pl.ds(i0 + k * IW, IW)], idx_v.at[b, k]
            )
            pltpu.async_copy(
                table_hbm.at[idx_v.at[b, k]],
                rows_v.at[b, k],
                gsem.at[b],
            )

    def wait_gathers(b):
        for k in range(KU):
            pltpu.make_async_copy(
                table_hbm.at[pl.ds(0, IW)], rows_v.at[b, k], gsem.at[b]
            ).wait()

    def wait_writeback(b):
        for k in range(KU):
            pltpu.make_async_copy(
                buf2.at[b, k], out_hbm.at[0, :, k], wsem.at[b]
            ).wait()

    fire(0, 0)

    def body(j, carry):
        b = lax.rem(j, 2)
        nb = 1 - b

        @pl.when(j + 1 < NJ)
        def _prefetch():
            fire(j + 1, nb)

        wait_gathers(b)

        @pl.when(j >= 2)
        def _drain():
            wait_writeback(b)

        def tk(k, carry2):
            iota = lax.iota(jnp.int32, 16)
            rv = rows_v.at[b, k]
            for dt in range(D // 8):
                for half in range(2):
                    vals = []
                    for dr4 in range(4):
                        dr = half * 4 + dr4
                        col = jnp.full((16,), dt * 8 + dr, jnp.int32)
                        for q in range(IW // 16):
                            v = plsc.load_gather(rv, [q * 16 + iota, col])
                            vals.append((dr, q, v))
                    for dr, q, v in vals:
                        buf2[b, k, dt, pl.ds(dr * IW + q * 16, 16)] = v
            return carry2

        lax.fori_loop(0, KU, tk, 0)

        for k in range(KU):
            pltpu.async_copy(
                buf2.at[b, k],
                out_hbm.at[j, :, wid * KU + k],
                wsem.at[b],
            )
        return carry

    lax.fori_loop(0, NJ, body, 0)
    wait_writeback(0)
    wait_writeback(1)


@jax.jit
def _gather(data, idxt):
    mesh = plsc.VectorSubcoreMesh(
        core_axis_name="c", subcore_axis_name="s",
        num_cores=NC, num_subcores=NS,
    )
    k = pl.kernel(
        _gather_body,
        out_type=jax.ShapeDtypeStruct((NJ, D // 8, NR // IW, 8 * IW),
                                      jnp.float32),
        mesh=mesh,
        scratch_types=[
            pltpu.VMEM((2, KU, IW), jnp.int32),
            pltpu.VMEM((2, KU, IW, D), jnp.float32),
            pltpu.VMEM((2, KU, D // 8, 8 * IW), jnp.float32),
            pltpu.SemaphoreType.DMA((2,)),
            pltpu.SemaphoreType.DMA((2,)),
        ],
        compiler_params=pltpu.CompilerParams(
            use_tc_tiling_on_sc=False, needs_layout_passes=False
        ),
    )
    out4 = k(data, idxt)
    out6 = out4.reshape(NJ, D // 8, NR // IW, 8, IW)
    return jnp.transpose(out6, (2, 4, 0, 1, 3)).reshape(NR, NJ, D)


def kernel(data, indices):
    return _gather(data, indices.astype(jnp.int32).T)


# speedup vs baseline: 1.1838x; 1.0434x over previous
import jax
import jax.numpy as jnp
from jax import lax
from jax.experimental import pallas as pl
from jax.experimental.pallas import tpu as pltpu
from jax.experimental.pallas import tpu_sc as plsc

NC = 2
NS = 16
NW = NC * NS

D = 32
NR = 16384
NI = 50
ROWS_PER_W = NR // NW
K = 16
NCHUNK = ROWS_PER_W // K


def _gather_body(table_hbm, idx_hbm, out_hbm, idx_v, rows_v, gsem, wsem):
    c = lax.axis_index("c")
    s = lax.axis_index("s")
    wid = s * NC + c
    base_row = wid * ROWS_PER_W

    def fire(g, b):
        r0 = base_row + g * K
        pltpu.sync_copy(idx_hbm.at[pl.ds(r0, K)], idx_v.at[b])
        for j in range(K):
            pltpu.async_copy(
                table_hbm.at[idx_v.at[b, j]],
                rows_v.at[b, j],
                gsem.at[b],
            )

    def wait_gathers(b):
        pltpu.make_async_copy(
            out_hbm.at[pl.ds(0, K)], rows_v.at[b], gsem.at[b]
        ).wait()

    def wait_writeback(b):
        pltpu.make_async_copy(
            rows_v.at[b], out_hbm.at[pl.ds(0, K)], wsem.at[b]
        ).wait()

    fire(0, 0)

    def body(g, carry):
        b = lax.rem(g, 2)
        nb = 1 - b

        @pl.when(g + 1 < NCHUNK)
        def _prefetch():
            @pl.when(g >= 1)
            def _drain_prev():
                wait_writeback(nb)

            fire(g + 1, nb)

        wait_gathers(b)
        r0 = base_row + g * K
        pltpu.async_copy(
            rows_v.at[b], out_hbm.at[pl.ds(r0, K)], wsem.at[b]
        )
        return carry

    lax.fori_loop(0, NCHUNK, body, 0)
    wait_writeback(0)
    wait_writeback(1)


@jax.jit
def _gather(data, idx):
    mesh = plsc.VectorSubcoreMesh(
        core_axis_name="c", subcore_axis_name="s",
        num_cores=NC, num_subcores=NS,
    )
    k = pl.kernel(
        _gather_body,
        out_type=jax.ShapeDtypeStruct((NR, NI, D), jnp.float32),
        mesh=mesh,
        scratch_types=[
            pltpu.VMEM((2, K, NI), jnp.int32),
            pltpu.VMEM((2, K, NI, D), jnp.float32),
            pltpu.SemaphoreType.DMA((2,)),
            pltpu.SemaphoreType.DMA((2,)),
        ],
        compiler_params=pltpu.CompilerParams(use_tc_tiling_on_sc=False),
    )
    return k(data, idx)


def kernel(data, indices):
    return _gather(data, indices.astype(jnp.int32))
